# Initial kernel scaffold; baseline (speedup 1.0000x reference)
import jax
import jax.numpy as jnp
from jax.experimental import pallas as pl


def kernel(cls_score, label):
    raise NotImplementedError("write your pallas kernel here")


# speedup vs baseline: 71.5433x; 71.5433x over previous
import functools

import jax
import jax.numpy as jnp
from jax import lax
from jax.experimental import pallas as pl
from jax.experimental.pallas import tpu as pltpu
from jax.experimental.pallas import tpu_sc as plsc

C = 19
H = W = 512
B = 4
N = B * H * W
Q = 256
NC, NS, L = 2, 16, 16
NT = NC * NS
PIX_PER_TILE = N // NT
BLK = 512
NBLK = PIX_PER_TILE // BLK
NVEC = BLK // L
HROWS = C * Q
HWORDS = HROWS * L
PPB = N // B
TPB = NT // B
FG_ONE = 1 << 16


def _sc_hist_body(scores_hbm, labels_hbm, hist_hbm, sbuf, lbuf, hist_v):
    wid = lax.axis_index("s") * NC + lax.axis_index("c")
    b = wid // TPB
    pos = (wid % TPB) * PIX_PER_TILE
    gbase = b * PPB + pos
    lane = lax.iota(jnp.int32, L)

    def _zero(i, _):
        hist_v[pl.ds(i * L, L)] = jnp.zeros((L,), jnp.int32)
        return 0

    lax.fori_loop(0, HROWS, _zero, 0)

    def _block(blk, _):
        off = pos + blk * BLK
        pltpu.sync_copy(scores_hbm.at[pl.ds(b * C, C), pl.ds(off, BLK)], sbuf)
        pltpu.sync_copy(labels_hbm.at[pl.ds(gbase + blk * BLK, BLK)], lbuf)

        def _vec(v, _):
            sl = pl.ds(v * L, L)
            lbl = lbuf[sl]
            es = [jnp.exp(sbuf[c, sl]) for c in range(C)]
            tot = es[0]
            for c in range(1, C):
                tot = tot + es[c]
            r = 1.0 / tot
            for c in range(C):
                p = es[c] * r
                isfg = lbl == c
                e = jnp.where(isfg, 1.0 - p, p)
                qi = jnp.minimum((e * float(Q)).astype(jnp.int32), Q - 1)
                idx = (qi << 4) + (lane + c * Q * L)
                add = jnp.where(isfg, jnp.int32(1 + FG_ONE), jnp.int32(1))
                plsc.addupdate_scatter(hist_v, [idx], add)
            return 0

        lax.fori_loop(0, NVEC, _vec, 0)
        return 0

    lax.fori_loop(0, NBLK, _block, 0)
    pltpu.sync_copy(hist_v, hist_hbm.at[wid])


def _tc_finish_body(hist_ref, out_ref):
    x = hist_ref[...]
    cnt = (x & 0xFFFF).astype(jnp.float32)
    fgc = (x >> 16).astype(jnp.float32)
    t = jnp.sum(cnt, axis=2)
    s = jnp.sum(fgc, axis=2)
    m = (lax.broadcasted_iota(jnp.int32, (Q, Q), 0)
         >= lax.broadcasted_iota(jnp.int32, (Q, Q), 1)).astype(jnp.float32)
    inc_t = jnp.dot(t, m, preferred_element_type=jnp.float32)
    inc_s = jnp.dot(s, m, preferred_element_type=jnp.float32)
    str_t = inc_t - t
    str_s = inc_s - s
    gts = inc_s[:, 0:1]

    def jac(n, f):
        union = gts + n - f
        safe = jnp.where(union > 0, union, 1.0)
        return 1.0 - jnp.where(union > 0, (gts - f) / safe, 1.0)

    centers = (lax.broadcasted_iota(jnp.int32, (1, Q), 1).astype(jnp.float32)
               + 0.5) * (1.0 / Q)
    contrib = centers * (jac(inc_t, inc_s) - jac(str_t, str_s))
    loss_c = jnp.sum(contrib, axis=1, keepdims=True)
    present = (gts > 0).astype(jnp.float32)
    num = jnp.sum(present * loss_c)
    den = jnp.sum(present)
    out_ref[...] = jnp.full((1, 1), num / den, jnp.float32)


@functools.partial(
    pl.kernel,
    mesh=plsc.VectorSubcoreMesh(core_axis_name="c", subcore_axis_name="s"),
    out_type=jax.ShapeDtypeStruct((NT, HWORDS), jnp.int32),
    compiler_params=pltpu.CompilerParams(
        use_tc_tiling_on_sc=False, needs_layout_passes=False),
    scratch_types=[
        pltpu.VMEM((C, BLK), jnp.float32),
        pltpu.VMEM((BLK,), jnp.int32),
        pltpu.VMEM((HWORDS,), jnp.int32),
    ],
)
def _sc_hist(*args):
    _sc_hist_body(*args)


_tc_finish = pl.pallas_call(
    _tc_finish_body,
    out_shape=jax.ShapeDtypeStruct((1, 1), jnp.float32),
)


def kernel(cls_score, label):
    scores_r = cls_score.reshape(B * C, PPB)
    labels_r = label.reshape(N)
    hist = _sc_hist(scores_r, labels_r)
    h4 = hist.reshape(NT, C, Q, L).transpose(1, 2, 0, 3).reshape(C, Q, NT * L)
    out = _tc_finish(h4)
    return out[0, 0]


# speedup vs baseline: 83.0975x; 1.1615x over previous
import functools

import jax
import jax.numpy as jnp
from jax import lax
from jax.experimental import pallas as pl
from jax.experimental.pallas import tpu as pltpu
from jax.experimental.pallas import tpu_sc as plsc

C = 19
H = W = 512
B = 4
N = B * H * W
Q = 256
NC, NS, L = 2, 16, 16
NT = NC * NS
PIX_PER_TILE = N // NT
BLK = 512
NBLK = PIX_PER_TILE // BLK
NVEC = BLK // L
HROWS = C * Q
HWORDS = HROWS * L
PPB = N // B
TPB = NT // B
FG_ONE = 1 << 16


def _sc_hist_body(scores_hbm, labels_hbm, hist_hbm, sbuf, lbuf, hist_v):
    wid = lax.axis_index("s") * NC + lax.axis_index("c")
    b = wid // TPB
    pos = (wid % TPB) * PIX_PER_TILE
    gbase = b * PPB + pos
    lane = lax.iota(jnp.int32, L)

    def _zero(i, _):
        hist_v[pl.ds(i * L, L)] = jnp.zeros((L,), jnp.int32)
        return 0

    lax.fori_loop(0, HROWS, _zero, 0)

    def _block(blk, _):
        off = pos + blk * BLK
        pltpu.sync_copy(scores_hbm.at[pl.ds(b * C, C), pl.ds(off, BLK)], sbuf)
        pltpu.sync_copy(labels_hbm.at[pl.ds(gbase + blk * BLK, BLK)], lbuf)

        def _vec(v, _):
            sl = pl.ds(v * L, L)
            lbl = lbuf[sl]
            es = [jnp.exp(sbuf[c, sl]) for c in range(C)]
            tot = es[0]
            for c in range(1, C):
                tot = tot + es[c]
            r = 1.0 / tot
            for c in range(C):
                p = es[c] * r
                isfg = lbl == c
                e = jnp.where(isfg, 1.0 - p, p)
                qi = jnp.minimum((e * float(Q)).astype(jnp.int32), Q - 1)
                idx = (qi << 4) + (lane + c * Q * L)
                add = jnp.where(isfg, jnp.int32(1 + FG_ONE), jnp.int32(1))
                plsc.addupdate_scatter(hist_v, [idx], add)
            return 0

        lax.fori_loop(0, NVEC, _vec, 0)
        return 0

    lax.fori_loop(0, NBLK, _block, 0)
    pltpu.sync_copy(hist_v, hist_hbm.at[wid])


def _tc_finish_body(hist_ref, out_ref):
    x = hist_ref[...]
    cnt = (x & 0xFFFF).astype(jnp.float32)
    fgc = (x >> 16).astype(jnp.float32)
    yt = jnp.sum(cnt, axis=0)
    ys = jnp.sum(fgc, axis=0)
    jj = lax.broadcasted_iota(jnp.int32, (Q * L, Q), 0) >> 4
    qq = lax.broadcasted_iota(jnp.int32, (Q * L, Q), 1)
    m_inc = (jj >= qq).astype(jnp.float32)
    m_eq = (jj == qq).astype(jnp.float32)
    inc_t = jnp.dot(yt, m_inc, preferred_element_type=jnp.float32)
    inc_s = jnp.dot(ys, m_inc, preferred_element_type=jnp.float32)
    t = jnp.dot(yt, m_eq, preferred_element_type=jnp.float32)
    s = jnp.dot(ys, m_eq, preferred_element_type=jnp.float32)
    str_t = inc_t - t
    str_s = inc_s - s
    gts = inc_s[:, 0:1]

    def jac(n, f):
        union = gts + n - f
        safe = jnp.where(union > 0, union, 1.0)
        return 1.0 - jnp.where(union > 0, (gts - f) / safe, 1.0)

    centers = (lax.broadcasted_iota(jnp.int32, (1, Q), 1).astype(jnp.float32)
               + 0.5) * (1.0 / Q)
    contrib = centers * (jac(inc_t, inc_s) - jac(str_t, str_s))
    loss_c = jnp.sum(contrib, axis=1, keepdims=True)
    present = (gts > 0).astype(jnp.float32)
    num = jnp.sum(present * loss_c)
    den = jnp.sum(present)
    out_ref[...] = jnp.full((1, 1), num / den, jnp.float32)


@functools.partial(
    pl.kernel,
    mesh=plsc.VectorSubcoreMesh(core_axis_name="c", subcore_axis_name="s"),
    out_type=jax.ShapeDtypeStruct((NT, HWORDS), jnp.int32),
    compiler_params=pltpu.CompilerParams(
        use_tc_tiling_on_sc=False, needs_layout_passes=False),
    scratch_types=[
        pltpu.VMEM((C, BLK), jnp.float32),
        pltpu.VMEM((BLK,), jnp.int32),
        pltpu.VMEM((HWORDS,), jnp.int32),
    ],
)
def _sc_hist(*args):
    _sc_hist_body(*args)


_tc_finish = pl.pallas_call(
    _tc_finish_body,
    out_shape=jax.ShapeDtypeStruct((1, 1), jnp.float32),
)


def kernel(cls_score, label):
    scores_r = cls_score.reshape(B * C, PPB)
    labels_r = label.reshape(N)
    hist = _sc_hist(scores_r, labels_r)
    out = _tc_finish(hist.reshape(NT, C, Q * L))
    return out[0, 0]


# speedup vs baseline: 111.5994x; 1.3430x over previous
import functools

import jax
import jax.numpy as jnp
from jax import lax
from jax.experimental import pallas as pl
from jax.experimental.pallas import tpu as pltpu
from jax.experimental.pallas import tpu_sc as plsc

C = 19
H = W = 512
B = 4
N = B * H * W
Q = 256
NC, NS, L = 2, 16, 16
NT = NC * NS
PIX_PER_TILE = N // NT
BLK = 512
NBLK = PIX_PER_TILE // BLK
NVEC = BLK // L
HROWS = C * Q
HWORDS = HROWS * L
PPB = N // B
TPB = NT // B
FG_ONE = 1 << 16


UNROLL = 2


def _sc_hist_body(scores_hbm, labels_hbm, hist_hbm, sbuf, lbuf, hist_v,
                  ssem0, ssem1, lsem0, lsem1):
    wid = lax.axis_index("s") * NC + lax.axis_index("c")
    b = wid // TPB
    pos = (wid % TPB) * PIX_PER_TILE
    gbase = b * PPB + pos
    lane = lax.iota(jnp.int32, L)
    ssems = (ssem0, ssem1)
    lsems = (lsem0, lsem1)

    def _copies(blk, buf):
        off = pos + blk * BLK
        return (
            pltpu.make_async_copy(
                scores_hbm.at[pl.ds(b * C, C), pl.ds(off, BLK)],
                sbuf.at[buf], ssems[buf]),
            pltpu.make_async_copy(
                labels_hbm.at[pl.ds(gbase + blk * BLK, BLK)],
                lbuf.at[buf], lsems[buf]),
        )

    for cp in _copies(0, 0) + _copies(1, 1):
        cp.start()

    def _zero(i, _):
        hist_v[pl.ds(i * L, L)] = jnp.zeros((L,), jnp.int32)
        return 0

    lax.fori_loop(0, HROWS, _zero, 0)

    def _vec(v, buf):
        sl = pl.ds(v * L, L)
        lbl = lbuf[buf, sl]
        es = [jnp.exp(sbuf[buf, c, sl]) for c in range(C)]
        tot = es[0]
        for c in range(1, C):
            tot = tot + es[c]
        rq = float(Q) / tot
        for c in range(C):
            q0 = es[c] * rq
            isfg = lbl == c
            qf = jnp.where(isfg, float(Q) - q0, q0)
            qi = jnp.minimum(qf.astype(jnp.int32), Q - 1)
            idx = (qi << 4) + (lane + c * Q * L)
            add = jnp.where(isfg, jnp.int32(1 + FG_ONE), jnp.int32(1))
            plsc.addupdate_scatter(hist_v, [idx], add)

    def _pair(g, _):
        for buf in (0, 1):
            blk = g * 2 + buf
            for cp in _copies(blk, buf):
                cp.wait()

            def _vgrp(u, _u):
                for k in range(UNROLL):
                    _vec(u * UNROLL + k, buf)
                return 0

            lax.fori_loop(0, NVEC // UNROLL, _vgrp, 0)

            @pl.when(blk + 2 < NBLK)
            def _():
                for cp in _copies(blk + 2, buf):
                    cp.start()
        return 0

    lax.fori_loop(0, NBLK // 2, _pair, 0)
    pltpu.sync_copy(hist_v, hist_hbm.at[wid])


def _tc_finish_body(hist_ref, out_ref):
    x = hist_ref[...]
    cnt = (x & 0xFFFF).astype(jnp.float32)
    fgc = (x >> 16).astype(jnp.float32)
    yt = jnp.sum(cnt, axis=0)
    ys = jnp.sum(fgc, axis=0)
    jj = lax.broadcasted_iota(jnp.int32, (Q * L, Q), 0) >> 4
    qq = lax.broadcasted_iota(jnp.int32, (Q * L, Q), 1)
    m_inc = (jj >= qq).astype(jnp.float32)
    m_eq = (jj == qq).astype(jnp.float32)
    inc_t = jnp.dot(yt, m_inc, preferred_element_type=jnp.float32)
    inc_s = jnp.dot(ys, m_inc, preferred_element_type=jnp.float32)
    t = jnp.dot(yt, m_eq, preferred_element_type=jnp.float32)
    s = jnp.dot(ys, m_eq, preferred_element_type=jnp.float32)
    str_t = inc_t - t
    str_s = inc_s - s
    gts = inc_s[:, 0:1]

    def jac(n, f):
        union = gts + n - f
        safe = jnp.where(union > 0, union, 1.0)
        return 1.0 - jnp.where(union > 0, (gts - f) / safe, 1.0)

    centers = (lax.broadcasted_iota(jnp.int32, (1, Q), 1).astype(jnp.float32)
               + 0.5) * (1.0 / Q)
    contrib = centers * (jac(inc_t, inc_s) - jac(str_t, str_s))
    loss_c = jnp.sum(contrib, axis=1, keepdims=True)
    present = (gts > 0).astype(jnp.float32)
    num = jnp.sum(present * loss_c)
    den = jnp.sum(present)
    out_ref[...] = jnp.full((1, 1), num / den, jnp.float32)


@functools.partial(
    pl.kernel,
    mesh=plsc.VectorSubcoreMesh(core_axis_name="c", subcore_axis_name="s"),
    out_type=jax.ShapeDtypeStruct((NT, HWORDS), jnp.int32),
    compiler_params=pltpu.CompilerParams(
        use_tc_tiling_on_sc=False, needs_layout_passes=False),
    scratch_types=[
        pltpu.VMEM((2, C, BLK), jnp.float32),
        pltpu.VMEM((2, BLK), jnp.int32),
        pltpu.VMEM((HWORDS,), jnp.int32),
        pltpu.SemaphoreType.DMA,
        pltpu.SemaphoreType.DMA,
        pltpu.SemaphoreType.DMA,
        pltpu.SemaphoreType.DMA,
    ],
)
def _sc_hist(*args):
    _sc_hist_body(*args)


_tc_finish = pl.pallas_call(
    _tc_finish_body,
    out_shape=jax.ShapeDtypeStruct((1, 1), jnp.float32),
)


def kernel(cls_score, label):
    scores_r = cls_score.reshape(B * C, PPB)
    labels_r = label.reshape(N)
    hist = _sc_hist(scores_r, labels_r)
    out = _tc_finish(hist.reshape(NT, C, Q * L))
    return out[0, 0]
